# baseline (speedup 1.0000x reference)
import functools

import jax
import jax.numpy as jnp
from jax import lax
from jax.experimental import pallas as pl
from jax.experimental.pallas import tpu as pltpu
from jax.experimental.pallas import tpu_sc as plsc

N = 50000
E = 800000
T = 4
D = 64
Q = 16
NQ = 4
HID = 16
C = 2

NC = 2
NS = 16
BATCH = 128
K = 4
G = 98
NB = G * K
PER_TILE = NB * BATCH
E_PAD = NS * PER_TILE
DUMMY = N
N1 = 50176
RPT = N1 // NS
CHUNK = 112

BN = 2000


def _sc_round_body(table, g1, s1, g2, s2, z2, z1, sums, histn,
                   acc_e, acc_n, hist_e, hist_n, gbuf, sbuf, rows, cbuf,
                   ones_v, sem_g0, sem_g1, sem_s0, sem_s1, sem_h):
    sem_g = [sem_g0, sem_g1]
    sem_s = [sem_s0, sem_s1]
    c = lax.axis_index("c")
    s = lax.axis_index("s")
    base = s * RPT

    for i in range(BATCH // 16):
        ones_v[pl.ds(i * 16, 16)] = jnp.full((16,), 1.0, jnp.float32)

    def pass_loop(gather_src, gidx, sidx, dst, hist, do_hist):
        def load_idx(g, p):
            pltpu.sync_copy(gidx.at[pl.ds(g * K, K)], gbuf.at[p])
            pltpu.sync_copy(sidx.at[pl.ds(g * K, K)], sbuf.at[p])

        def fire_gathers(p):
            for b in range(K):
                pltpu.async_copy(gather_src.at[gbuf.at[p, b]],
                                 rows.at[p, b], sem_g[p])

        def wait_gathers(p):
            for b in range(K):
                pltpu.make_async_copy(gather_src.at[gbuf.at[p, b]],
                                      rows.at[p, b], sem_g[p]).wait()

        def fire_scatters(p):
            for b in range(K):
                pltpu.async_copy(rows.at[p, b], dst.at[sbuf.at[p, b]],
                                 sem_s[p], add=True)
            if do_hist is not None:
                @pl.when(do_hist)
                def _():
                    counts = []
                    for b in range(K):
                        counts.append(
                            pltpu.async_copy(ones_v, hist.at[sbuf.at[p, b]],
                                             sem_h, add=True))
                    for h in counts:
                        h.wait()

        def wait_scatters(p):
            for b in range(K):
                pltpu.make_async_copy(rows.at[p, b], dst.at[sbuf.at[p, b]],
                                      sem_s[p]).wait()

        def half(g, p, first=False, fire_next=True):
            wait_gathers(p)
            fire_scatters(p)
            if not first:
                wait_scatters(1 - p)
            if fire_next:
                load_idx(g + 1, 1 - p)
                fire_gathers(1 - p)

        load_idx(0, 0)
        fire_gathers(0)
        half(0, 0, first=True)

        def steady(i, carry):
            half(2 * i + 1, 1)
            half(2 * i + 2, 0)
            return carry

        lax.fori_loop(0, (G - 2) // 2, steady, 0)
        half(G - 1, 1, fire_next=False)
        wait_scatters(1)

    for ph in range(2):
        pltpu.sync_copy(z2, acc_e.at[pl.ds(base, RPT)])
        pltpu.sync_copy(z2, acc_n.at[pl.ds(base, RPT)])
        if ph == 0:
            pltpu.sync_copy(z1, hist_e.at[pl.ds(base, RPT)])

            @pl.when(c == 0)
            def _():
                pltpu.sync_copy(z1, hist_n.at[pl.ds(base, RPT)])
        plsc.subcore_barrier()

        pass_loop(table, g1.at[2 * ph + c, s], s1.at[s], acc_e, hist_e,
                  jnp.bool_(True) if ph == 0 else None)
        plsc.subcore_barrier()

        def chunk_step(j, carry):
            off = base + j * CHUNK
            pltpu.sync_copy(acc_e.at[pl.ds(off, CHUNK)],
                            rows.at[0, 0, pl.ds(0, CHUNK)])
            pltpu.sync_copy(hist_e.at[pl.ds(off, CHUNK)],
                            cbuf.at[pl.ds(0, CHUNK)])
            for i2 in range(CHUNK // 16):
                v = cbuf[pl.ds(i2 * 16, 16)]
                cbuf[pl.ds(i2 * 16, 16)] = 1.0 / jnp.maximum(v, 1.0)

            def row_step(i2, carry2):
                rv = cbuf[pl.ds(i2 * 16, 16)]
                for j in range(16):
                    i = i2 * 16 + j
                    rows[0, 0, i, :] = rows[0, 0, i, :] * rv[j]
                return carry2

            lax.fori_loop(0, CHUNK // 16, row_step, 0)
            pltpu.sync_copy(rows.at[0, 0, pl.ds(0, CHUNK)],
                            acc_e.at[pl.ds(off, CHUNK)])
            return carry

        lax.fori_loop(0, RPT // CHUNK, chunk_step, 0)
        plsc.subcore_barrier()

        pass_loop(acc_e, g2.at[s], s2.at[s], acc_n, hist_n,
                  (c == 0) if ph == 0 else None)
        plsc.subcore_barrier()

        pltpu.sync_copy(acc_n.at[pl.ds(base, RPT)],
                        sums.at[2 * ph + c, pl.ds(base, RPT)])
        if ph == 0:
            @pl.when(c == 0)
            def _():
                pltpu.sync_copy(hist_n.at[pl.ds(base, RPT)],
                                histn.at[pl.ds(base, RPT)])
        plsc.subcore_barrier()


_sc_round = functools.partial(
    pl.kernel,
    out_type=(
        jax.ShapeDtypeStruct((NQ, N1, Q), jnp.float32),
        jax.ShapeDtypeStruct((N1,), jnp.float32),
    ),
    mesh=plsc.VectorSubcoreMesh(core_axis_name="c", subcore_axis_name="s"),
    compiler_params=pltpu.CompilerParams(use_tc_tiling_on_sc=False),
    scratch_types=[
        pltpu.VMEM_SHARED((N1, Q), jnp.float32),
        pltpu.VMEM_SHARED((N1, Q), jnp.float32),
        pltpu.VMEM_SHARED((N1,), jnp.float32),
        pltpu.VMEM_SHARED((N1,), jnp.float32),
        pltpu.VMEM((2, K, BATCH), jnp.int32),
        pltpu.VMEM((2, K, BATCH), jnp.int32),
        pltpu.VMEM((2, K, BATCH, Q), jnp.float32),
        pltpu.VMEM((BATCH,), jnp.float32),
        pltpu.VMEM((BATCH,), jnp.float32),
        pltpu.SemaphoreType.DMA,
        pltpu.SemaphoreType.DMA,
        pltpu.SemaphoreType.DMA,
        pltpu.SemaphoreType.DMA,
        pltpu.SemaphoreType.DMA,
    ],
)(_sc_round_body)


def _node_body(sums_ref, cnt_ref, w_ref, b_ref, out_ref, flat_ref):
    x = jnp.concatenate(
        [sums_ref[0], sums_ref[1], sums_ref[2], sums_ref[3]], axis=1)
    cnt = cnt_ref[...]
    r = 1.0 / jnp.maximum(cnt, 1.0)
    ind = jnp.where(cnt > 0.0, 1.0, 0.0)
    y = jnp.dot(x * r, w_ref[...], preferred_element_type=jnp.float32)
    h = jnp.maximum(y + b_ref[...] * ind, 0.0)
    for q in range(NQ):
        out_ref[q] = h[:, q * Q:(q + 1) * Q]
    flat_ref[...] = h


def _node_call(sums, cnt, w_t, b):
    return pl.pallas_call(
        _node_body,
        grid=(N // BN,),
        in_specs=[
            pl.BlockSpec((NQ, BN, Q), lambda i: (0, i, 0)),
            pl.BlockSpec((BN, 1), lambda i: (i, 0)),
            pl.BlockSpec((D, D), lambda i: (0, 0)),
            pl.BlockSpec((1, D), lambda i: (0, 0)),
        ],
        out_specs=[
            pl.BlockSpec((NQ, BN, Q), lambda i: (0, i, 0)),
            pl.BlockSpec((BN, D), lambda i: (i, 0)),
        ],
        out_shape=[
            jax.ShapeDtypeStruct((NQ, N, Q), jnp.float32),
            jax.ShapeDtypeStruct((N, D), jnp.float32),
        ],
    )(sums, cnt, w_t, b)


def _lstm_body(xs_ref, wi0_ref, wh0_ref, b0_ref, wi1_ref, wh1_ref, b1_ref,
               wo_ref, bo_ref, out_ref):
    h1 = jnp.zeros((BN, HID), jnp.float32)
    c1 = jnp.zeros((BN, HID), jnp.float32)
    h2 = jnp.zeros((BN, HID), jnp.float32)
    c2 = jnp.zeros((BN, HID), jnp.float32)

    def cell(x, h, c, wi, wh, b):
        g = (jnp.dot(x, wi, preferred_element_type=jnp.float32)
             + jnp.dot(h, wh, preferred_element_type=jnp.float32) + b)
        i = jax.nn.sigmoid(g[:, 0 * HID:1 * HID])
        f = jax.nn.sigmoid(g[:, 1 * HID:2 * HID])
        gg = jnp.tanh(g[:, 2 * HID:3 * HID])
        o = jax.nn.sigmoid(g[:, 3 * HID:4 * HID])
        c_new = f * c + i * gg
        h_new = o * jnp.tanh(c_new)
        return h_new, c_new

    for t in range(T):
        h1, c1 = cell(xs_ref[t], h1, c1, wi0_ref[...], wh0_ref[...],
                      b0_ref[...])
        h2, c2 = cell(h1, h2, c2, wi1_ref[...], wh1_ref[...], b1_ref[...])
        out_ref[t] = (jnp.dot(h2, wo_ref[...],
                              preferred_element_type=jnp.float32)
                      + bo_ref[...])


def _lstm_call(xs, wi0_t, wh0_t, b0, wi1_t, wh1_t, b1, wo_t, bo):
    return pl.pallas_call(
        _lstm_body,
        grid=(N // BN,),
        in_specs=[
            pl.BlockSpec((T, BN, D), lambda i: (0, i, 0)),
            pl.BlockSpec((D, 4 * HID), lambda i: (0, 0)),
            pl.BlockSpec((HID, 4 * HID), lambda i: (0, 0)),
            pl.BlockSpec((1, 4 * HID), lambda i: (0, 0)),
            pl.BlockSpec((HID, 4 * HID), lambda i: (0, 0)),
            pl.BlockSpec((HID, 4 * HID), lambda i: (0, 0)),
            pl.BlockSpec((1, 4 * HID), lambda i: (0, 0)),
            pl.BlockSpec((HID, C), lambda i: (0, 0)),
            pl.BlockSpec((1, C), lambda i: (0, 0)),
        ],
        out_specs=pl.BlockSpec((T, BN, C), lambda i: (0, i, 0)),
        out_shape=jax.ShapeDtypeStruct((T, N, C), jnp.float32),
    )(xs, wi0_t, wh0_t, b0, wi1_t, wh1_t, b1, wo_t, bo)


def kernel(hyperedge_seq, W_emb, b_emb, W_e2n, b_e2n, W_ih0, W_hh0, b_ih0,
           b_hh0, W_ih1, W_hh1, b_ih1, b_hh1, W_out, b_out):
    hs = hyperedge_seq.astype(jnp.int32)

    pad_g = jnp.zeros((E_PAD - E,), jnp.int32)
    pad_s = jnp.full((E_PAD - E,), DUMMY, jnp.int32)

    def prep(src, eidx):
        g1 = jnp.concatenate([src, pad_g]).reshape(NS, NB, BATCH)
        g1 = jnp.stack([g1 + q * N for q in range(NQ)])
        s1 = jnp.concatenate([eidx, pad_s]).reshape(NS, NB, BATCH)
        g2 = jnp.concatenate([eidx, pad_g]).reshape(NS, NB, BATCH)
        s2 = jnp.concatenate([src, pad_s]).reshape(NS, NB, BATCH)
        return g1, s1, g2, s2

    z2 = jnp.zeros((RPT, Q), jnp.float32)
    z1 = jnp.zeros((RPT,), jnp.float32)

    h0 = W_emb + b_emb
    quarters = jnp.stack([h0[:, q * Q:(q + 1) * Q] for q in range(NQ)])

    w_e2n_t = W_e2n.T
    b_e2n_r = b_e2n.reshape(1, D)

    xs_list = []
    for t in range(T):
        g1, s1, g2, s2 = prep(hs[t, 0], hs[t, 1])
        sums_n, hist_n = _sc_round(quarters.reshape(NQ * N, Q), g1, s1, g2,
                                   s2, z2, z1)
        quarters, flat = _node_call(sums_n[:, :N, :], hist_n[:N].reshape(N, 1),
                                    w_e2n_t, b_e2n_r)
        xs_list.append(flat)

    xs = jnp.stack(xs_list)
    out = _lstm_call(xs,
                     W_ih0.T, W_hh0.T, (b_ih0 + b_hh0).reshape(1, 4 * HID),
                     W_ih1.T, W_hh1.T, (b_ih1 + b_hh1).reshape(1, 4 * HID),
                     W_out.T, b_out.reshape(1, C))
    return out


# speedup vs baseline: 1.2465x; 1.2465x over previous
import functools

import jax
import jax.numpy as jnp
from jax import lax
from jax.experimental import pallas as pl
from jax.experimental.pallas import tpu as pltpu
from jax.experimental.pallas import tpu_sc as plsc

N = 50000
E = 800000
T = 4
D = 64
Q = 16
NQ = 4
HID = 16
C = 2

NC = 2
NS = 16
BATCH = 128
K = 8
G = 49
NB = G * K
PER_TILE = NB * BATCH
E_PAD = NS * PER_TILE
DUMMY = N
N1 = 50176
RPT = N1 // NS
CHUNK = 112

BN = 2000


def _sc_round_body(table, g1, s1, g2, s2, z2, z1, sums, histn,
                   acc_e, acc_n, hist_e, hist_n, gbuf, sbuf, rows, cbuf,
                   ones_v, sem_g0, sem_g1, sem_s0, sem_s1, sem_h):
    sem_g = [sem_g0, sem_g1]
    sem_s = [sem_s0, sem_s1]
    c = lax.axis_index("c")
    s = lax.axis_index("s")
    base = s * RPT

    for i in range(BATCH // 16):
        ones_v[pl.ds(i * 16, 16)] = jnp.full((16,), 1.0, jnp.float32)

    def pass_loop(gather_src, gidx, sidx, dst, hist, do_hist):
        def load_idx(g, p):
            pltpu.sync_copy(gidx.at[pl.ds(g * K, K)], gbuf.at[p])
            pltpu.sync_copy(sidx.at[pl.ds(g * K, K)], sbuf.at[p])

        def group(g, p, prefetch=True):
            gathers = []
            for b in range(K):
                gathers.append(
                    pltpu.async_copy(gather_src.at[gbuf.at[p, b]],
                                     rows.at[b], sem_g0))
            if prefetch:
                load_idx(g + 1, 1 - p)
            if do_hist is not None:
                @pl.when(do_hist)
                def _():
                    for b in range(K):
                        pltpu.async_copy(ones_v, hist.at[sbuf.at[p, b]],
                                         sem_h, add=True)
            for h in gathers:
                h.wait()
            scatters = []
            for b in range(K):
                scatters.append(
                    pltpu.async_copy(rows.at[b], dst.at[sbuf.at[p, b]],
                                     sem_s0, add=True))
            for h in scatters:
                h.wait()
            if do_hist is not None:
                @pl.when(do_hist)
                def _():
                    for b in range(K):
                        pltpu.make_async_copy(ones_v, hist.at[sbuf.at[p, b]],
                                              sem_h).wait()

        load_idx(0, 0)

        def steady(i, carry):
            group(2 * i, 0)
            group(2 * i + 1, 1)
            return carry

        lax.fori_loop(0, (G - 1) // 2, steady, 0)
        group(G - 1, 0, prefetch=False)

    for ph in range(2):
        pltpu.sync_copy(z2, acc_e.at[pl.ds(base, RPT)])
        pltpu.sync_copy(z2, acc_n.at[pl.ds(base, RPT)])
        if ph == 0:
            pltpu.sync_copy(z1, hist_e.at[pl.ds(base, RPT)])

            @pl.when(c == 0)
            def _():
                pltpu.sync_copy(z1, hist_n.at[pl.ds(base, RPT)])
        plsc.subcore_barrier()

        pass_loop(table, g1.at[2 * ph + c, s], s1.at[s], acc_e, hist_e,
                  jnp.bool_(True) if ph == 0 else None)
        plsc.subcore_barrier()

        def chunk_step(j, carry):
            off = base + j * CHUNK
            pltpu.sync_copy(acc_e.at[pl.ds(off, CHUNK)],
                            rows.at[0, pl.ds(0, CHUNK)])
            pltpu.sync_copy(hist_e.at[pl.ds(off, CHUNK)],
                            cbuf.at[pl.ds(0, CHUNK)])
            for i2 in range(CHUNK // 16):
                v = cbuf[pl.ds(i2 * 16, 16)]
                cbuf[pl.ds(i2 * 16, 16)] = 1.0 / jnp.maximum(v, 1.0)

            def row_step(i2, carry2):
                rv = cbuf[pl.ds(i2 * 16, 16)]
                for j in range(16):
                    i = i2 * 16 + j
                    rows[0, i, :] = rows[0, i, :] * rv[j]
                return carry2

            lax.fori_loop(0, CHUNK // 16, row_step, 0)
            pltpu.sync_copy(rows.at[0, pl.ds(0, CHUNK)],
                            acc_e.at[pl.ds(off, CHUNK)])
            return carry

        lax.fori_loop(0, RPT // CHUNK, chunk_step, 0)
        plsc.subcore_barrier()

        pass_loop(acc_e, g2.at[s], s2.at[s], acc_n, hist_n,
                  (c == 0) if ph == 0 else None)
        plsc.subcore_barrier()

        pltpu.sync_copy(acc_n.at[pl.ds(base, RPT)],
                        sums.at[2 * ph + c, pl.ds(base, RPT)])
        if ph == 0:
            @pl.when(c == 0)
            def _():
                pltpu.sync_copy(hist_n.at[pl.ds(base, RPT)],
                                histn.at[pl.ds(base, RPT)])
        plsc.subcore_barrier()


_sc_round = functools.partial(
    pl.kernel,
    out_type=(
        jax.ShapeDtypeStruct((NQ, N1, Q), jnp.float32),
        jax.ShapeDtypeStruct((N1,), jnp.float32),
    ),
    mesh=plsc.VectorSubcoreMesh(core_axis_name="c", subcore_axis_name="s"),
    compiler_params=pltpu.CompilerParams(use_tc_tiling_on_sc=False),
    scratch_types=[
        pltpu.VMEM_SHARED((N1, Q), jnp.float32),
        pltpu.VMEM_SHARED((N1, Q), jnp.float32),
        pltpu.VMEM_SHARED((N1,), jnp.float32),
        pltpu.VMEM_SHARED((N1,), jnp.float32),
        pltpu.VMEM((2, K, BATCH), jnp.int32),
        pltpu.VMEM((2, K, BATCH), jnp.int32),
        pltpu.VMEM((K, BATCH, Q), jnp.float32),
        pltpu.VMEM((BATCH,), jnp.float32),
        pltpu.VMEM((BATCH,), jnp.float32),
        pltpu.SemaphoreType.DMA,
        pltpu.SemaphoreType.DMA,
        pltpu.SemaphoreType.DMA,
        pltpu.SemaphoreType.DMA,
        pltpu.SemaphoreType.DMA,
    ],
)(_sc_round_body)


def _node_body(sums_ref, cnt_ref, w_ref, b_ref, out_ref, flat_ref):
    x = jnp.concatenate(
        [sums_ref[0], sums_ref[1], sums_ref[2], sums_ref[3]], axis=1)
    cnt = cnt_ref[...]
    r = 1.0 / jnp.maximum(cnt, 1.0)
    ind = jnp.where(cnt > 0.0, 1.0, 0.0)
    y = jnp.dot(x * r, w_ref[...], preferred_element_type=jnp.float32)
    h = jnp.maximum(y + b_ref[...] * ind, 0.0)
    for q in range(NQ):
        out_ref[q] = h[:, q * Q:(q + 1) * Q]
    flat_ref[...] = h


def _node_call(sums, cnt, w_t, b):
    return pl.pallas_call(
        _node_body,
        grid=(N // BN,),
        in_specs=[
            pl.BlockSpec((NQ, BN, Q), lambda i: (0, i, 0)),
            pl.BlockSpec((BN, 1), lambda i: (i, 0)),
            pl.BlockSpec((D, D), lambda i: (0, 0)),
            pl.BlockSpec((1, D), lambda i: (0, 0)),
        ],
        out_specs=[
            pl.BlockSpec((NQ, BN, Q), lambda i: (0, i, 0)),
            pl.BlockSpec((BN, D), lambda i: (i, 0)),
        ],
        out_shape=[
            jax.ShapeDtypeStruct((NQ, N, Q), jnp.float32),
            jax.ShapeDtypeStruct((N, D), jnp.float32),
        ],
    )(sums, cnt, w_t, b)


def _lstm_body(xs_ref, wi0_ref, wh0_ref, b0_ref, wi1_ref, wh1_ref, b1_ref,
               wo_ref, bo_ref, out_ref):
    h1 = jnp.zeros((BN, HID), jnp.float32)
    c1 = jnp.zeros((BN, HID), jnp.float32)
    h2 = jnp.zeros((BN, HID), jnp.float32)
    c2 = jnp.zeros((BN, HID), jnp.float32)

    def cell(x, h, c, wi, wh, b):
        g = (jnp.dot(x, wi, preferred_element_type=jnp.float32)
             + jnp.dot(h, wh, preferred_element_type=jnp.float32) + b)
        i = jax.nn.sigmoid(g[:, 0 * HID:1 * HID])
        f = jax.nn.sigmoid(g[:, 1 * HID:2 * HID])
        gg = jnp.tanh(g[:, 2 * HID:3 * HID])
        o = jax.nn.sigmoid(g[:, 3 * HID:4 * HID])
        c_new = f * c + i * gg
        h_new = o * jnp.tanh(c_new)
        return h_new, c_new

    for t in range(T):
        h1, c1 = cell(xs_ref[t], h1, c1, wi0_ref[...], wh0_ref[...],
                      b0_ref[...])
        h2, c2 = cell(h1, h2, c2, wi1_ref[...], wh1_ref[...], b1_ref[...])
        out_ref[t] = (jnp.dot(h2, wo_ref[...],
                              preferred_element_type=jnp.float32)
                      + bo_ref[...])


def _lstm_call(xs, wi0_t, wh0_t, b0, wi1_t, wh1_t, b1, wo_t, bo):
    return pl.pallas_call(
        _lstm_body,
        grid=(N // BN,),
        in_specs=[
            pl.BlockSpec((T, BN, D), lambda i: (0, i, 0)),
            pl.BlockSpec((D, 4 * HID), lambda i: (0, 0)),
            pl.BlockSpec((HID, 4 * HID), lambda i: (0, 0)),
            pl.BlockSpec((1, 4 * HID), lambda i: (0, 0)),
            pl.BlockSpec((HID, 4 * HID), lambda i: (0, 0)),
            pl.BlockSpec((HID, 4 * HID), lambda i: (0, 0)),
            pl.BlockSpec((1, 4 * HID), lambda i: (0, 0)),
            pl.BlockSpec((HID, C), lambda i: (0, 0)),
            pl.BlockSpec((1, C), lambda i: (0, 0)),
        ],
        out_specs=pl.BlockSpec((T, BN, C), lambda i: (0, i, 0)),
        out_shape=jax.ShapeDtypeStruct((T, N, C), jnp.float32),
    )(xs, wi0_t, wh0_t, b0, wi1_t, wh1_t, b1, wo_t, bo)


def kernel(hyperedge_seq, W_emb, b_emb, W_e2n, b_e2n, W_ih0, W_hh0, b_ih0,
           b_hh0, W_ih1, W_hh1, b_ih1, b_hh1, W_out, b_out):
    hs = hyperedge_seq.astype(jnp.int32)

    pad_g = jnp.zeros((E_PAD - E,), jnp.int32)
    pad_s = jnp.full((E_PAD - E,), DUMMY, jnp.int32)

    def prep(src, eidx):
        g1 = jnp.concatenate([src, pad_g]).reshape(NS, NB, BATCH)
        g1 = jnp.stack([g1 + q * N for q in range(NQ)])
        s1 = jnp.concatenate([eidx, pad_s]).reshape(NS, NB, BATCH)
        g2 = jnp.concatenate([eidx, pad_g]).reshape(NS, NB, BATCH)
        s2 = jnp.concatenate([src, pad_s]).reshape(NS, NB, BATCH)
        return g1, s1, g2, s2

    z2 = jnp.zeros((RPT, Q), jnp.float32)
    z1 = jnp.zeros((RPT,), jnp.float32)

    h0 = W_emb + b_emb
    quarters = jnp.stack([h0[:, q * Q:(q + 1) * Q] for q in range(NQ)])

    w_e2n_t = W_e2n.T
    b_e2n_r = b_e2n.reshape(1, D)

    xs_list = []
    for t in range(T):
        g1, s1, g2, s2 = prep(hs[t, 0], hs[t, 1])
        sums_n, hist_n = _sc_round(quarters.reshape(NQ * N, Q), g1, s1, g2,
                                   s2, z2, z1)
        quarters, flat = _node_call(sums_n[:, :N, :], hist_n[:N].reshape(N, 1),
                                    w_e2n_t, b_e2n_r)
        xs_list.append(flat)

    xs = jnp.stack(xs_list)
    out = _lstm_call(xs,
                     W_ih0.T, W_hh0.T, (b_ih0 + b_hh0).reshape(1, 4 * HID),
                     W_ih1.T, W_hh1.T, (b_ih1 + b_hh1).reshape(1, 4 * HID),
                     W_out.T, b_out.reshape(1, C))
    return out
